# baseline (speedup 1.0000x reference)
import functools

import jax
import jax.numpy as jnp
from jax import lax
from jax.experimental import pallas as pl
from jax.experimental.pallas import tpu as pltpu
from jax.experimental.pallas import tpu_sc as plsc

_NC = 2
_NS = 16
_NW = _NC * _NS

_BATCH = 4096
_SEQ = 200
_DIM = 64
_CHUNK = 100
_NCHUNK = _SEQ // _CHUNK
_BPW = _BATCH // _NW
_LPR = _DIM // 16


def _sc_body(idx_hbm, emb_hbm, out_hbm, idx_v, buf_v, out_v, sem0, sem1):
    sems = (sem0, sem1)
    wid = lax.axis_index("s") * _NC + lax.axis_index("c")

    pltpu.sync_copy(idx_hbm.at[pl.ds(wid * _BPW * _NCHUNK, _BPW * _NCHUNK)],
                    idx_v)

    def start_gathers(b, slot):
        for c in range(_NCHUNK):
            pltpu.async_copy(
                emb_hbm.at[idx_v.at[b * _NCHUNK + c]],
                buf_v.at[slot, pl.ds(c * _CHUNK, _CHUNK)],
                sems[slot])

    def drain(slot):
        pltpu.make_async_copy(emb_hbm.at[pl.ds(0, _SEQ)],
                              buf_v.at[slot], sems[slot]).wait()

    def accumulate(b, slot):
        buf = buf_v.at[slot]

        def rows(i, acc):
            for u in range(8):
                r = i * 8 + u
                acc = tuple(
                    acc[l] + buf[r, pl.ds(16 * l, 16)] for l in range(_LPR))
            return acc

        zero = jnp.zeros((16,), jnp.float32)
        acc = plsc.parallel_loop(0, _SEQ // 8, 1, unroll=2,
                                 carry=(zero,) * _LPR)(rows)
        for l in range(_LPR):
            out_v[b, pl.ds(16 * l, 16)] = acc[l] * (1.0 / _SEQ)

    start_gathers(0, 0)

    def loop_body(b0):
        for p in range(2):
            b = b0 + p

            @pl.when(b + 1 < _BPW)
            def _():
                start_gathers(b + 1, 1 - p)

            drain(p)
            accumulate(b, p)

    pl.loop(0, _BPW, step=2)(loop_body)

    pltpu.sync_copy(out_v, out_hbm.at[pl.ds(wid * _BPW, _BPW)])


@functools.partial(
    pl.kernel,
    out_type=jax.ShapeDtypeStruct((_BATCH, _DIM), jnp.float32),
    mesh=plsc.VectorSubcoreMesh(core_axis_name="c", subcore_axis_name="s"),
    scratch_types=[
        pltpu.VMEM((_BPW * _NCHUNK, _CHUNK), jnp.int32),
        pltpu.VMEM((2, _SEQ, _DIM), jnp.float32),
        pltpu.VMEM((_BPW, _DIM), jnp.float32),
        pltpu.SemaphoreType.DMA,
        pltpu.SemaphoreType.DMA,
    ],
    compiler_params=pltpu.CompilerParams(use_tc_tiling_on_sc=False),
)
def _pooled_lookup(idx_hbm, emb_hbm, out_hbm, idx_v, buf_v, out_v, s0, s1):
    _sc_body(idx_hbm, emb_hbm, out_hbm, idx_v, buf_v, out_v, s0, s1)


@jax.jit
def kernel(text_tokens, emb):
    idx = text_tokens.astype(jnp.int32).reshape(_BATCH * _NCHUNK, _CHUNK)
    return _pooled_lookup(idx, emb)


# speedup vs baseline: 1.0035x; 1.0035x over previous
import functools

import jax
import jax.numpy as jnp
from jax import lax
from jax.experimental import pallas as pl
from jax.experimental.pallas import tpu as pltpu
from jax.experimental.pallas import tpu_sc as plsc

_NC = 2
_NS = 16
_NW = _NC * _NS

_BATCH = 4096
_SEQ = 200
_DIM = 64
_CHUNKS = (104, 96)
_BPW = _BATCH // _NW
_LPR = _DIM // 16


def _sc_body(idx_hbm, emb_hbm, out_hbm, idx_v, buf_v, out_v, sem0, sem1):
    sems = (sem0, sem1)
    wid = lax.axis_index("s") * _NC + lax.axis_index("c")

    pltpu.sync_copy(idx_hbm.at[pl.ds(wid * _BPW, _BPW)], idx_v)

    def start_gathers(b, slot):
        off = 0
        for n in _CHUNKS:
            pltpu.async_copy(
                emb_hbm.at[idx_v.at[b, pl.ds(off, n)]],
                buf_v.at[slot, pl.ds(off, n)],
                sems[slot])
            off += n

    def drain(slot):
        pltpu.make_async_copy(emb_hbm.at[pl.ds(0, _SEQ)],
                              buf_v.at[slot], sems[slot]).wait()

    def accumulate(b, slot):
        buf = buf_v.at[slot]

        def rows(i, acc):
            for u in range(8):
                r = i * 8 + u
                acc = tuple(
                    acc[l] + buf[r, pl.ds(16 * l, 16)] for l in range(_LPR))
            return acc

        zero = jnp.zeros((16,), jnp.float32)
        acc = plsc.parallel_loop(0, _SEQ // 8, 1, unroll=2,
                                 carry=(zero,) * _LPR)(rows)
        for l in range(_LPR):
            out_v[b, pl.ds(16 * l, 16)] = acc[l] * (1.0 / _SEQ)

    start_gathers(0, 0)

    def loop_body(b0):
        for p in range(2):
            b = b0 + p

            @pl.when(b + 1 < _BPW)
            def _():
                start_gathers(b + 1, 1 - p)

            drain(p)
            accumulate(b, p)

    pl.loop(0, _BPW, step=2)(loop_body)

    pltpu.sync_copy(out_v, out_hbm.at[pl.ds(wid * _BPW, _BPW)])


@functools.partial(
    pl.kernel,
    out_type=jax.ShapeDtypeStruct((_BATCH, _DIM), jnp.float32),
    mesh=plsc.VectorSubcoreMesh(core_axis_name="c", subcore_axis_name="s"),
    scratch_types=[
        pltpu.VMEM((_BPW, _SEQ), jnp.int32),
        pltpu.VMEM((2, _SEQ, _DIM), jnp.float32),
        pltpu.VMEM((_BPW, _DIM), jnp.float32),
        pltpu.SemaphoreType.DMA,
        pltpu.SemaphoreType.DMA,
    ],
    compiler_params=pltpu.CompilerParams(use_tc_tiling_on_sc=False),
)
def _pooled_lookup(idx_hbm, emb_hbm, out_hbm, idx_v, buf_v, out_v, s0, s1):
    _sc_body(idx_hbm, emb_hbm, out_hbm, idx_v, buf_v, out_v, s0, s1)


@jax.jit
def kernel(text_tokens, emb):
    return _pooled_lookup(text_tokens, emb)
